# baseline (speedup 1.0000x reference)
import functools

import jax
import jax.numpy as jnp
from jax import lax
from jax.experimental import pallas as pl
from jax.experimental.pallas import tpu as pltpu
from jax.experimental.pallas import tpu_sc as plsc

_N = 10000
_E = 320000
_D = 128
_NCLS = 256

_NC = 2
_NS = 16
_NW = _NC * _NS
_B = 64
_NB = 160
_DB = 128
_NBD = 80
_EPW = _B * _NB
_EPAD = _NW * _EPW
_NDUMMY = 240
_NROWS = _N + _NDUMMY
_RPS = _NROWS // _NS
_BLK = 1024
_GRID = _NROWS // _BLK

_mesh = plsc.VectorSubcoreMesh(
    core_axis_name="c", subcore_axis_name="s", num_cores=_NC, num_subcores=_NS
)


@functools.partial(
    pl.kernel,
    out_type=jax.ShapeDtypeStruct((_NC, _NROWS), jnp.float32),
    mesh=_mesh,
    scratch_types=[
        pltpu.VMEM((_NBD, _DB), jnp.int32),
        pltpu.VMEM((_RPS,), jnp.float32),
        pltpu.VMEM((_DB,), jnp.float32),
        pltpu.VMEM_SHARED((_NROWS,), jnp.float32),
    ],
)
def _deg_kernel(dst_hbm, out_hbm, dst_v, zb_v, ones_v, deg_sp):
    cid = lax.axis_index("c")
    sid = lax.axis_index("s")
    wid = cid * _NS + sid

    def fill_z(i, _):
        zb_v[pl.ds(i * 16, 16)] = jnp.zeros((16,), jnp.float32)
        return _

    lax.fori_loop(0, _RPS // 16, fill_z, None)
    for i in range(_DB // 16):
        ones_v[pl.ds(i * 16, 16)] = jnp.ones((16,), jnp.float32)

    pltpu.sync_copy(dst_hbm.at[wid], dst_v)
    pltpu.sync_copy(zb_v, deg_sp.at[pl.ds(sid * _RPS, _RPS)])
    plsc.subcore_barrier()

    def body(j, _):
        pltpu.sync_copy(ones_v, deg_sp.at[dst_v.at[j]], add=True)
        return _

    lax.fori_loop(0, _NBD, body, None)
    plsc.subcore_barrier()
    pltpu.sync_copy(
        deg_sp.at[pl.ds(sid * _RPS, _RPS)],
        out_hbm.at[cid, pl.ds(sid * _RPS, _RPS)],
    )


_SCAT_SCRATCH = [
        pltpu.VMEM((_B,), jnp.int32),
        pltpu.VMEM((_B,), jnp.int32),
        pltpu.VMEM((_B,), jnp.int32),
        pltpu.VMEM((_B,), jnp.int32),
        pltpu.VMEM((4, _B), jnp.int32),
        pltpu.VMEM((4, _B), jnp.int32),
        pltpu.VMEM((4, _B, _D), jnp.float32),
        pltpu.SemaphoreType.DMA,
        pltpu.SemaphoreType.DMA,
        pltpu.SemaphoreType.DMA,
        pltpu.SemaphoreType.DMA,
        pltpu.SemaphoreType.DMA,
        pltpu.SemaphoreType.DMA,
        pltpu.SemaphoreType.DMA,
        pltpu.SemaphoreType.DMA,
        pltpu.SemaphoreType.DMA,
        pltpu.SemaphoreType.DMA,
        pltpu.SemaphoreType.DMA,
        pltpu.SemaphoreType.DMA,
        pltpu.VMEM_SHARED((_NROWS, _D), jnp.float32),
]


def _scat_body(hp_hbm, se_hbm, out_hbm, se0, se1, se2, se3, srcb, dstb, rb,
               i0, i1, i2, i3, g0, g1, g2, g3, s0, s1, s2, s3, y_sp):
    cid = lax.axis_index("c")
    sid = lax.axis_index("s")
    wid = cid * _NS + sid
    seb = (se0, se1, se2, se3)
    isem = (i0, i1, i2, i3)
    gsem = (g0, g1, g2, g3)
    ssem = (s0, s1, s2, s3)

    def fill_z(i, _):
        for k in range(_D // 16):
            rb[0, i, pl.ds(k * 16, 16)] = jnp.zeros((16,), jnp.float32)
        return _

    lax.fori_loop(0, _B, fill_z, None)

    def zcopy(j, _):
        pltpu.sync_copy(rb.at[0], y_sp.at[pl.ds(sid * _RPS + j * _B, _B)])
        return _

    lax.fori_loop(0, _RPS // _B, zcopy, None)

    plsc.subcore_barrier()

    def ifetch(k, slot):
        pass

    def iwait(k, slot):
        pltpu.sync_copy(se_hbm.at[wid * (_NB + 4) + k], seb[slot])

    def unpack(slot):
        for c in range(_B // 16):
            w = seb[slot][pl.ds(c * 16, 16)]
            srcb[slot, pl.ds(c * 16, 16)] = w & 0xFFFF
            dstb[slot, pl.ds(c * 16, 16)] = lax.shift_right_logical(w, 16)

    def gissue(slot):
        pltpu.async_copy(hp_hbm.at[srcb.at[slot]], rb.at[slot], gsem[slot])

    def gwait(slot):
        pltpu.make_async_copy(
            hp_hbm.at[srcb.at[slot]], rb.at[slot], gsem[slot]
        ).wait()

    def sissue(slot):
        pltpu.sync_copy(rb.at[slot], y_sp.at[dstb.at[slot]], add=True)

    def swait(slot):
        pass

    def step(k, slot, first, last):
        if not first:
            swait((slot + 2) % 4)
        if not last:
            iwait(k + 2, (slot + 2) % 4)
            unpack((slot + 2) % 4)
            gissue((slot + 2) % 4)
            ifetch(k + 4, slot)
        gwait(slot)
        sissue(slot)

    for k in range(4):
        ifetch(k, k)
    for k in range(2):
        iwait(k, k)
        unpack(k)
        gissue(k)
    step(0, 0, True, False)
    step(1, 1, True, False)

    def mid(t, _):
        for i in range(4):
            step(4 * t + 2 + i, (2 + i) % 4, False, False)
        return _

    lax.fori_loop(0, (_NB - 4) // 4, mid, None)
    step(_NB - 2, (_NB - 2) % 4, False, True)
    step(_NB - 1, (_NB - 1) % 4, False, True)
    swait((_NB - 2) % 4)
    swait((_NB - 1) % 4)

    plsc.subcore_barrier()

    def dump(j, _):
        pltpu.sync_copy(
            y_sp.at[pl.ds(sid * _RPS + j * 128, 128)],
            out_hbm.at[cid, pl.ds(sid * _RPS + j * 128, 128)],
        )
        return _

    lax.fori_loop(0, _RPS // 128, dump, None)


_scat_kernel = pl.kernel(
    _scat_body,
    out_type=jax.ShapeDtypeStruct((_NC, _NROWS, _D), jnp.float32),
    mesh=_mesh,
    scratch_types=_SCAT_SCRATCH,
)


def _k1_body(x_ref, w1_ref, degp_ref, hp_ref, d_ref):
    deg = degp_ref[0, :] + degp_ref[1, :] + 1.0
    d = lax.rsqrt(deg)
    h = jnp.dot(x_ref[...], w1_ref[...], preferred_element_type=jnp.float32)
    hp_ref[...] = h * d[:, None]
    d_ref[...] = d[:, None]


def _tc_k1(x_pad, w1, deg_parts):
    return pl.pallas_call(
        _k1_body,
        grid=(_GRID,),
        in_specs=[
            pl.BlockSpec((_BLK, _D), lambda i: (i, 0)),
            pl.BlockSpec((_D, _D), lambda i: (0, 0)),
            pl.BlockSpec((_NC, _BLK), lambda i: (0, i)),
        ],
        out_specs=[
            pl.BlockSpec((_BLK, _D), lambda i: (i, 0)),
            pl.BlockSpec((_BLK, 1), lambda i: (i, 0)),
        ],
        out_shape=[
            jax.ShapeDtypeStruct((_NROWS, _D), jnp.float32),
            jax.ShapeDtypeStruct((_NROWS, 1), jnp.float32),
        ],
    )(x_pad, w1, deg_parts)


def _mid_body(yp_ref, hp_ref, d_ref, b1_ref, w2_ref, hp2_ref):
    ysum = yp_ref[0] + yp_ref[1] + hp_ref[...]
    out1 = ysum * d_ref[...] + b1_ref[...]
    h = jnp.where(out1 > 0.0, out1, jnp.exp(out1) - 1.0)
    h2 = jnp.dot(h, w2_ref[...], preferred_element_type=jnp.float32)
    hp2_ref[...] = h2 * d_ref[...]


def _tc_mid(y_parts, hp, d, b1, w2):
    return pl.pallas_call(
        _mid_body,
        grid=(_GRID,),
        in_specs=[
            pl.BlockSpec((_NC, _BLK, _D), lambda i: (0, i, 0)),
            pl.BlockSpec((_BLK, _D), lambda i: (i, 0)),
            pl.BlockSpec((_BLK, 1), lambda i: (i, 0)),
            pl.BlockSpec((1, _D), lambda i: (0, 0)),
            pl.BlockSpec((_D, _D), lambda i: (0, 0)),
        ],
        out_specs=pl.BlockSpec((_BLK, _D), lambda i: (i, 0)),
        out_shape=jax.ShapeDtypeStruct((_NROWS, _D), jnp.float32),
    )(y_parts, hp, d, b1, w2)


def _fin_body(yp_ref, hp2_ref, d_ref, b2_ref, wc_ref, bc_ref, emb_ref,
              logit_ref):
    ysum = yp_ref[0] + yp_ref[1] + hp2_ref[...]
    emb = ysum * d_ref[...] + b2_ref[...]
    emb_ref[...] = emb
    logit_ref[...] = (
        jnp.dot(emb, wc_ref[...], preferred_element_type=jnp.float32)
        + bc_ref[...]
    )


def _tc_fin(y2_parts, hp2, d, b2, wc, bc):
    return pl.pallas_call(
        _fin_body,
        grid=(_GRID,),
        in_specs=[
            pl.BlockSpec((_NC, _BLK, _D), lambda i: (0, i, 0)),
            pl.BlockSpec((_BLK, _D), lambda i: (i, 0)),
            pl.BlockSpec((_BLK, 1), lambda i: (i, 0)),
            pl.BlockSpec((1, _D), lambda i: (0, 0)),
            pl.BlockSpec((_D, _NCLS), lambda i: (0, 0)),
            pl.BlockSpec((1, _NCLS), lambda i: (0, 0)),
        ],
        out_specs=[
            pl.BlockSpec((_BLK, _D), lambda i: (i, 0)),
            pl.BlockSpec((_BLK, _NCLS), lambda i: (i, 0)),
        ],
        out_shape=[
            jax.ShapeDtypeStruct((_NROWS, _D), jnp.float32),
            jax.ShapeDtypeStruct((_NROWS, _NCLS), jnp.float32),
        ],
    )(y2_parts, hp2, d, b2, wc, bc)


def kernel(x, edge_index, W1, b1, W2, b2, Wc, bc):
    src = edge_index[0]
    dst = edge_index[1]
    pad = _EPAD - _E
    ar = jnp.arange(pad, dtype=jnp.int32)
    src_p = jnp.concatenate([src, ar % _N])
    dst_p = jnp.concatenate([dst, _N + ar % _NDUMMY])
    se_p = (src_p | (dst_p << 16)).reshape(_NW, _NB, _B)
    se_p = jnp.pad(se_p, ((0, 0), (0, 4), (0, 0))).reshape(-1, _B)
    dst_deg = dst_p.reshape(_NW, _NBD, _DB)
    x_pad = jnp.concatenate(
        [x, jnp.zeros((_NROWS - _N, _D), jnp.float32)], axis=0
    )

    deg_parts = _deg_kernel(dst_deg)
    hp, d = _tc_k1(x_pad, W1, deg_parts)
    y1 = _scat_kernel(hp, se_p)
    hp2 = _tc_mid(y1, hp, d, b1.reshape(1, _D), W2)
    y2 = _scat_kernel(hp2, se_p)
    emb, logits = _tc_fin(y2, hp2, d, b2.reshape(1, _D), Wc, bc.reshape(1, _NCLS))
    return emb[:_N], logits[:_N]


# speedup vs baseline: 1.3125x; 1.3125x over previous
import functools

import jax
import jax.numpy as jnp
from jax import lax
from jax.experimental import pallas as pl
from jax.experimental.pallas import tpu as pltpu
from jax.experimental.pallas import tpu_sc as plsc

_N = 10000
_E = 320000
_D = 128
_NCLS = 256

_NC = 2
_NS = 16
_NW = _NC * _NS
_B = 64
_NB = 160
_DB = 128
_NBD = 80
_EPW = _B * _NB
_EPAD = _NW * _EPW
_NDUMMY = 112
_NROWS = _N + _NDUMMY
_RPS = _NROWS // _NS
_NRD = 10240
_RPD = _NRD // _NS
_BLK = 1264
_GRID = _NROWS // _BLK

_mesh = plsc.VectorSubcoreMesh(
    core_axis_name="c", subcore_axis_name="s", num_cores=_NC, num_subcores=_NS
)


@functools.partial(
    pl.kernel,
    out_type=jax.ShapeDtypeStruct((_NC, _NRD), jnp.float32),
    mesh=_mesh,
    scratch_types=[
        pltpu.VMEM((_NBD, _DB), jnp.int32),
        pltpu.VMEM((_RPD,), jnp.float32),
        pltpu.VMEM((_DB,), jnp.float32),
        pltpu.VMEM_SHARED((_NRD,), jnp.float32),
    ],
)
def _deg_kernel(dst_hbm, out_hbm, dst_v, zb_v, ones_v, deg_sp):
    cid = lax.axis_index("c")
    sid = lax.axis_index("s")
    wid = cid * _NS + sid

    def fill_z(i, _):
        zb_v[pl.ds(i * 16, 16)] = jnp.zeros((16,), jnp.float32)
        return _

    lax.fori_loop(0, _RPD // 16, fill_z, None)
    for i in range(_DB // 16):
        ones_v[pl.ds(i * 16, 16)] = jnp.ones((16,), jnp.float32)

    pltpu.sync_copy(dst_hbm.at[wid], dst_v)
    pltpu.sync_copy(zb_v, deg_sp.at[pl.ds(sid * _RPD, _RPD)])
    plsc.subcore_barrier()

    def body(j, _):
        pltpu.sync_copy(ones_v, deg_sp.at[dst_v.at[j]], add=True)
        return _

    lax.fori_loop(0, _NBD, body, None)
    plsc.subcore_barrier()
    pltpu.sync_copy(
        deg_sp.at[pl.ds(sid * _RPD, _RPD)],
        out_hbm.at[cid, pl.ds(sid * _RPD, _RPD)],
    )


_SCAT_SCRATCH = [
        pltpu.VMEM((_NB // 2, _B), jnp.int32),
        pltpu.VMEM((4, _B), jnp.int32),
        pltpu.VMEM((4, _B), jnp.int32),
        pltpu.VMEM((4, _B, _D), jnp.float32),
        pltpu.SemaphoreType.DMA,
        pltpu.SemaphoreType.DMA,
        pltpu.SemaphoreType.DMA,
        pltpu.SemaphoreType.DMA,
        pltpu.SemaphoreType.DMA,
        pltpu.SemaphoreType.DMA,
        pltpu.SemaphoreType.DMA,
        pltpu.SemaphoreType.DMA,
        pltpu.VMEM_SHARED((_NROWS, _D), jnp.float32),
]


def _scat_body(hp_hbm, se_hbm, out_hbm, se_v, srcb, dstb, rb,
               g0, g1, g2, g3, s0, s1, s2, s3, y_sp):
    cid = lax.axis_index("c")
    sid = lax.axis_index("s")
    wid = cid * _NS + sid
    gsem = (g0, g1, g2, g3)
    ssem = (s0, s1, s2, s3)

    def fill_z(i, _):
        for k in range(_D // 16):
            rb[0, i, pl.ds(k * 16, 16)] = jnp.zeros((16,), jnp.float32)
        return _

    lax.fori_loop(0, _B, fill_z, None)

    def zcopy(j, _):
        pltpu.sync_copy(rb.at[0], y_sp.at[pl.ds(sid * _RPS + j * _B, _B)])
        return _

    lax.fori_loop(0, 9, zcopy, None)
    pltpu.sync_copy(rb.at[0], y_sp.at[pl.ds(sid * _RPS + _RPS - _B, _B)])

    pltpu.sync_copy(se_hbm.at[wid, pl.ds(0, _NB // 2)], se_v)
    plsc.subcore_barrier()

    def unpack(row, slot):
        for c in range(_B // 16):
            w = se_v[row, pl.ds(c * 16, 16)]
            srcb[slot, pl.ds(c * 16, 16)] = w & 0xFFFF
            dstb[slot, pl.ds(c * 16, 16)] = lax.shift_right_logical(w, 16)

    def gissue(slot):
        pltpu.async_copy(hp_hbm.at[srcb.at[slot]], rb.at[slot], gsem[slot])

    def gwait(slot):
        pltpu.make_async_copy(
            hp_hbm.at[srcb.at[slot]], rb.at[slot], gsem[slot]
        ).wait()

    def sissue(slot):
        pltpu.async_copy(
            rb.at[slot], y_sp.at[dstb.at[slot]], ssem[slot], add=True
        )

    def swait(slot):
        pltpu.make_async_copy(
            rb.at[slot], y_sp.at[dstb.at[slot]], ssem[slot]
        ).wait()

    def step(k, slot, ofs, first=False, last=False):
        if not first:
            swait((slot + 2) % 4)
        if not last:
            unpack(k + 2 - ofs, (slot + 2) % 4)
            gissue((slot + 2) % 4)
        gwait(slot)
        sissue(slot)

    for k in range(2):
        unpack(k, k)
        gissue(k)
    step(0, 0, 0, first=True)
    step(1, 1, 0, first=True)

    def mid1(t, _):
        for i in range(4):
            step(4 * t + 2 + i, (2 + i) % 4, 0)
        return _

    lax.fori_loop(0, 19, mid1, None)
    pltpu.sync_copy(se_hbm.at[wid, pl.ds(_NB // 2, _NB // 2)], se_v)

    def mid2(t, _):
        for i in range(4):
            step(78 + 4 * t + i, (2 + i) % 4, 80)
        return _

    lax.fori_loop(0, 20, mid2, None)
    step(_NB - 2, (_NB - 2) % 4, 0, last=True)
    step(_NB - 1, (_NB - 1) % 4, 0, last=True)
    swait((_NB - 2) % 4)
    swait((_NB - 1) % 4)

    plsc.subcore_barrier()

    def dump(j, _):
        pltpu.sync_copy(
            y_sp.at[pl.ds(sid * _RPS + j * 128, 128)],
            out_hbm.at[cid, pl.ds(sid * _RPS + j * 128, 128)],
        )
        return _

    lax.fori_loop(0, 4, dump, None)
    pltpu.sync_copy(
        y_sp.at[pl.ds(sid * _RPS + _RPS - 128, 128)],
        out_hbm.at[cid, pl.ds(sid * _RPS + _RPS - 128, 128)],
    )


_scat_kernel = pl.kernel(
    _scat_body,
    out_type=jax.ShapeDtypeStruct((_NC, _NROWS, _D), jnp.float32),
    mesh=_mesh,
    scratch_types=_SCAT_SCRATCH,
)


def _k1_body(x_ref, w1_ref, degp_ref, hp_ref, d_ref):
    deg = degp_ref[:, 0] + degp_ref[:, 1] + 1.0
    d = lax.rsqrt(deg)
    h = jnp.dot(x_ref[...], w1_ref[...], preferred_element_type=jnp.float32)
    hp_ref[...] = h * d[:, None]
    d_ref[...] = d[:, None]


def _tc_k1(x_pad, w1, deg_parts):
    return pl.pallas_call(
        _k1_body,
        grid=(_GRID,),
        in_specs=[
            pl.BlockSpec((_BLK, _D), lambda i: (i, 0)),
            pl.BlockSpec((_D, _D), lambda i: (0, 0)),
            pl.BlockSpec((_BLK, _NC), lambda i: (i, 0)),
        ],
        out_specs=[
            pl.BlockSpec((_BLK, _D), lambda i: (i, 0)),
            pl.BlockSpec((_BLK, 1), lambda i: (i, 0)),
        ],
        out_shape=[
            jax.ShapeDtypeStruct((_NROWS, _D), jnp.float32),
            jax.ShapeDtypeStruct((_NROWS, 1), jnp.float32),
        ],
    )(x_pad, w1, deg_parts)


def _mid_body(yp_ref, hp_ref, d_ref, b1_ref, w2_ref, hp2_ref):
    ysum = yp_ref[0] + yp_ref[1] + hp_ref[...]
    out1 = ysum * d_ref[...] + b1_ref[...]
    h = jnp.where(out1 > 0.0, out1, jnp.exp(out1) - 1.0)
    h2 = jnp.dot(h, w2_ref[...], preferred_element_type=jnp.float32)
    hp2_ref[...] = h2 * d_ref[...]


def _tc_mid(y_parts, hp, d, b1, w2):
    return pl.pallas_call(
        _mid_body,
        grid=(_GRID,),
        in_specs=[
            pl.BlockSpec((_NC, _BLK, _D), lambda i: (0, i, 0)),
            pl.BlockSpec((_BLK, _D), lambda i: (i, 0)),
            pl.BlockSpec((_BLK, 1), lambda i: (i, 0)),
            pl.BlockSpec((1, _D), lambda i: (0, 0)),
            pl.BlockSpec((_D, _D), lambda i: (0, 0)),
        ],
        out_specs=pl.BlockSpec((_BLK, _D), lambda i: (i, 0)),
        out_shape=jax.ShapeDtypeStruct((_NROWS, _D), jnp.float32),
    )(y_parts, hp, d, b1, w2)


def _fin_body(yp_ref, hp2_ref, d_ref, b2_ref, wc_ref, bc_ref, emb_ref,
              logit_ref):
    ysum = yp_ref[0] + yp_ref[1] + hp2_ref[...]
    emb = ysum * d_ref[...] + b2_ref[...]
    emb_ref[...] = emb
    logit_ref[...] = (
        jnp.dot(emb, wc_ref[...], preferred_element_type=jnp.float32)
        + bc_ref[...]
    )


def _tc_fin(y2_parts, hp2, d, b2, wc, bc):
    return pl.pallas_call(
        _fin_body,
        grid=(_GRID,),
        in_specs=[
            pl.BlockSpec((_NC, _BLK, _D), lambda i: (0, i, 0)),
            pl.BlockSpec((_BLK, _D), lambda i: (i, 0)),
            pl.BlockSpec((_BLK, 1), lambda i: (i, 0)),
            pl.BlockSpec((1, _D), lambda i: (0, 0)),
            pl.BlockSpec((_D, _NCLS), lambda i: (0, 0)),
            pl.BlockSpec((1, _NCLS), lambda i: (0, 0)),
        ],
        out_specs=[
            pl.BlockSpec((_BLK, _D), lambda i: (i, 0)),
            pl.BlockSpec((_BLK, _NCLS), lambda i: (i, 0)),
        ],
        out_shape=[
            jax.ShapeDtypeStruct((_NROWS, _D), jnp.float32),
            jax.ShapeDtypeStruct((_NROWS, _NCLS), jnp.float32),
        ],
    )(y2_parts, hp2, d, b2, wc, bc)


def kernel(x, edge_index, W1, b1, W2, b2, Wc, bc):
    src = edge_index[0]
    dst = edge_index[1]
    pad = _EPAD - _E
    ar = jnp.arange(pad, dtype=jnp.int32)
    src_p = jnp.concatenate([src, ar % _N])
    dst_p = jnp.concatenate([dst, _N + ar % _NDUMMY])
    se_p = (src_p | (dst_p << 16)).reshape(_NW, _NB, _B)
    dst_deg = dst_p.reshape(_NW, _NBD, _DB)
    x_pad = jnp.concatenate(
        [x, jnp.zeros((_NROWS - _N, _D), jnp.float32)], axis=0
    )

    deg_parts = _deg_kernel(dst_deg)
    hp, d = _tc_k1(x_pad, W1, deg_parts.T[:_NROWS])
    y1 = _scat_kernel(hp, se_p)
    hp2 = _tc_mid(y1, hp, d, b1.reshape(1, _D), W2)
    y2 = _scat_kernel(hp2, se_p)
    emb, logits = _tc_fin(y2, hp2, d, b2.reshape(1, _D), Wc, bc.reshape(1, _NCLS))
    return emb[:_N], logits[:_N]


# speedup vs baseline: 1.4518x; 1.1061x over previous
import functools

import jax
import jax.numpy as jnp
from jax import lax
from jax.experimental import pallas as pl
from jax.experimental.pallas import tpu as pltpu
from jax.experimental.pallas import tpu_sc as plsc

_N = 10000
_E = 320000
_D = 128
_NCLS = 256

_NC = 2
_NS = 16
_NW = _NC * _NS
_B = 64
_NB = 160
_DB = 128
_NBD = 80
_EPW = _B * _NB
_EPAD = _NW * _EPW
_NDUMMY = 240
_NROWS = _N + _NDUMMY
_RPS = _NROWS // _NS
_BLK = 2000
_GRID = _N // _BLK

_mesh = plsc.VectorSubcoreMesh(
    core_axis_name="c", subcore_axis_name="s", num_cores=_NC, num_subcores=_NS
)


@functools.partial(
    pl.kernel,
    out_type=jax.ShapeDtypeStruct((_NC, _NROWS), jnp.float32),
    mesh=_mesh,
    scratch_types=[
        pltpu.VMEM((_NBD, _DB), jnp.int32),
        pltpu.VMEM((_RPS,), jnp.float32),
        pltpu.VMEM((_DB,), jnp.float32),
        pltpu.SemaphoreType.DMA,
        pltpu.VMEM_SHARED((_NROWS,), jnp.float32),
    ],
)
def _deg_kernel(dst_hbm, out_hbm, dst_v, zb_v, ones_v, dsem, deg_sp):
    cid = lax.axis_index("c")
    sid = lax.axis_index("s")
    wid = cid * _NS + sid

    pltpu.async_copy(dst_hbm.at[wid], dst_v, dsem)

    def fill_z(i, _):
        zb_v[pl.ds(i * 16, 16)] = jnp.zeros((16,), jnp.float32)
        return _

    lax.fori_loop(0, _RPS // 16, fill_z, None)
    for i in range(_DB // 16):
        ones_v[pl.ds(i * 16, 16)] = jnp.ones((16,), jnp.float32)

    pltpu.make_async_copy(dst_hbm.at[wid], dst_v, dsem).wait()
    pltpu.sync_copy(zb_v, deg_sp.at[pl.ds(sid * _RPS, _RPS)])
    plsc.subcore_barrier()

    def group(g, _):
        for i in range(8):
            pltpu.async_copy(
                ones_v, deg_sp.at[dst_v.at[g * 8 + i]], dsem, add=True
            )
        for i in range(8):
            pltpu.make_async_copy(
                ones_v, deg_sp.at[dst_v.at[g * 8 + i]], dsem
            ).wait()
        return _

    lax.fori_loop(0, _NBD // 8, group, None)
    plsc.subcore_barrier()
    pltpu.sync_copy(
        deg_sp.at[pl.ds(sid * _RPS, _RPS)],
        out_hbm.at[cid, pl.ds(sid * _RPS, _RPS)],
    )


_SCAT_SCRATCH = [
    pltpu.VMEM((_NB, _B), jnp.int32),
    pltpu.VMEM((3, _B), jnp.int32),
    pltpu.VMEM((3, _B), jnp.int32),
    pltpu.VMEM((3, _B, _D), jnp.float32),
    pltpu.SemaphoreType.DMA,
    pltpu.SemaphoreType.DMA,
    pltpu.SemaphoreType.DMA,
    pltpu.SemaphoreType.DMA,
    pltpu.SemaphoreType.DMA,
    pltpu.SemaphoreType.DMA,
    pltpu.VMEM_SHARED((_NROWS, _D), jnp.float32),
]


def _scat_body(hp_hbm, se_hbm, out_hbm, se_v, srcb, dstb, rb,
               g0, g1, g2, s0, s1, s2, y_sp):
    cid = lax.axis_index("c")
    sid = lax.axis_index("s")
    wid = cid * _NS + sid
    gsem = (g0, g1, g2)
    ssem = (s0, s1, s2)

    pltpu.async_copy(se_hbm.at[wid], se_v, g0)

    def fill_z(i, _):
        for k in range(_D // 16):
            rb[0, i, pl.ds(k * 16, 16)] = jnp.zeros((16,), jnp.float32)
        return _

    lax.fori_loop(0, _B, fill_z, None)

    for j in range(_RPS // _B):
        pltpu.async_copy(
            rb.at[0], y_sp.at[pl.ds(sid * _RPS + j * _B, _B)], s0
        )
    for j in range(_RPS // _B):
        pltpu.make_async_copy(
            rb.at[0], y_sp.at[pl.ds(sid * _RPS + j * _B, _B)], s0
        ).wait()
    pltpu.make_async_copy(se_hbm.at[wid], se_v, g0).wait()
    plsc.subcore_barrier()

    def unpack(k, slot):
        for c in range(_B // 16):
            w = se_v[k, pl.ds(c * 16, 16)]
            srcb[slot, pl.ds(c * 16, 16)] = w & 0xFFFF
            dstb[slot, pl.ds(c * 16, 16)] = lax.shift_right_logical(w, 16)

    def gissue(slot):
        pltpu.async_copy(hp_hbm.at[srcb.at[slot]], rb.at[slot], gsem[slot])

    def gwait(slot):
        pltpu.make_async_copy(
            hp_hbm.at[srcb.at[slot]], rb.at[slot], gsem[slot]
        ).wait()

    def sissue(slot):
        pltpu.async_copy(
            rb.at[slot], y_sp.at[dstb.at[slot]], ssem[slot], add=True
        )

    def swait(slot):
        pltpu.make_async_copy(
            rb.at[slot], y_sp.at[dstb.at[slot]], ssem[slot]
        ).wait()

    def step(k, slot, first, last):
        if not first:
            swait((slot + 2) % 3)
        if not last:
            unpack(k + 2, (slot + 2) % 3)
            gissue((slot + 2) % 3)
        gwait(slot)
        sissue(slot)

    unpack(0, 0)
    gissue(0)
    unpack(1, 1)
    gissue(1)
    step(0, 0, True, False)
    step(1, 1, False, False)

    def mid(t, _):
        for i in range(3):
            step(3 * t + 2 + i, (2 + i) % 3, False, False)
        return _

    lax.fori_loop(0, (_NB - 4) // 3, mid, None)
    step(_NB - 2, (_NB - 2) % 3, False, True)
    step(_NB - 1, (_NB - 1) % 3, False, True)
    swait((_NB - 1) % 3)

    plsc.subcore_barrier()

    for j in range(_RPS // 128):
        pltpu.async_copy(
            y_sp.at[pl.ds(sid * _RPS + j * 128, 128)],
            out_hbm.at[cid, pl.ds(sid * _RPS + j * 128, 128)],
            s0,
        )
    for j in range(_RPS // 128):
        pltpu.make_async_copy(
            y_sp.at[pl.ds(sid * _RPS + j * 128, 128)],
            out_hbm.at[cid, pl.ds(sid * _RPS + j * 128, 128)],
            s0,
        ).wait()


_scat_kernel = pl.kernel(
    _scat_body,
    out_type=jax.ShapeDtypeStruct((_NC, _NROWS, _D), jnp.float32),
    mesh=_mesh,
    scratch_types=_SCAT_SCRATCH,
)


def _k1_body(x_ref, w1_ref, degp_ref, hp_ref, d_ref):
    deg = degp_ref[:, 0] + degp_ref[:, 1] + 1.0
    d = lax.rsqrt(deg)
    h = jnp.dot(x_ref[...], w1_ref[...], preferred_element_type=jnp.float32)
    hp_ref[...] = h * d[:, None]
    d_ref[...] = d[:, None]


def _tc_k1(x, w1, degt):
    return pl.pallas_call(
        _k1_body,
        grid=(_GRID,),
        in_specs=[
            pl.BlockSpec((_BLK, _D), lambda i: (i, 0)),
            pl.BlockSpec((_D, _D), lambda i: (0, 0)),
            pl.BlockSpec((_BLK, _NC), lambda i: (i, 0)),
        ],
        out_specs=[
            pl.BlockSpec((_BLK, _D), lambda i: (i, 0)),
            pl.BlockSpec((_BLK, 1), lambda i: (i, 0)),
        ],
        out_shape=[
            jax.ShapeDtypeStruct((_N, _D), jnp.float32),
            jax.ShapeDtypeStruct((_N, 1), jnp.float32),
        ],
    )(x, w1, degt)


def _mid_body(yp_ref, hp_ref, d_ref, b1_ref, w2_ref, hp2_ref):
    ysum = yp_ref[0] + yp_ref[1] + hp_ref[...]
    out1 = ysum * d_ref[...] + b1_ref[...]
    h = jnp.where(out1 > 0.0, out1, jnp.exp(out1) - 1.0)
    h2 = jnp.dot(h, w2_ref[...], preferred_element_type=jnp.float32)
    hp2_ref[...] = h2 * d_ref[...]


def _tc_mid(y_parts, hp, d, b1, w2):
    return pl.pallas_call(
        _mid_body,
        grid=(_GRID,),
        in_specs=[
            pl.BlockSpec((_NC, _BLK, _D), lambda i: (0, i, 0)),
            pl.BlockSpec((_BLK, _D), lambda i: (i, 0)),
            pl.BlockSpec((_BLK, 1), lambda i: (i, 0)),
            pl.BlockSpec((1, _D), lambda i: (0, 0)),
            pl.BlockSpec((_D, _D), lambda i: (0, 0)),
        ],
        out_specs=pl.BlockSpec((_BLK, _D), lambda i: (i, 0)),
        out_shape=jax.ShapeDtypeStruct((_N, _D), jnp.float32),
    )(y_parts, hp, d, b1, w2)


def _fin_body(yp_ref, hp2_ref, d_ref, b2_ref, wc_ref, bc_ref, emb_ref,
              logit_ref):
    ysum = yp_ref[0] + yp_ref[1] + hp2_ref[...]
    emb = ysum * d_ref[...] + b2_ref[...]
    emb_ref[...] = emb
    logit_ref[...] = (
        jnp.dot(emb, wc_ref[...], preferred_element_type=jnp.float32)
        + bc_ref[...]
    )


def _tc_fin(y2_parts, hp2, d, b2, wc, bc):
    return pl.pallas_call(
        _fin_body,
        grid=(_GRID,),
        in_specs=[
            pl.BlockSpec((_NC, _BLK, _D), lambda i: (0, i, 0)),
            pl.BlockSpec((_BLK, _D), lambda i: (i, 0)),
            pl.BlockSpec((_BLK, 1), lambda i: (i, 0)),
            pl.BlockSpec((1, _D), lambda i: (0, 0)),
            pl.BlockSpec((_D, _NCLS), lambda i: (0, 0)),
            pl.BlockSpec((1, _NCLS), lambda i: (0, 0)),
        ],
        out_specs=[
            pl.BlockSpec((_BLK, _D), lambda i: (i, 0)),
            pl.BlockSpec((_BLK, _NCLS), lambda i: (i, 0)),
        ],
        out_shape=[
            jax.ShapeDtypeStruct((_N, _D), jnp.float32),
            jax.ShapeDtypeStruct((_N, _NCLS), jnp.float32),
        ],
    )(y2_parts, hp2, d, b2, wc, bc)


def kernel(x, edge_index, W1, b1, W2, b2, Wc, bc):
    src = edge_index[0]
    dst = edge_index[1]
    pad = _EPAD - _E
    ar = jnp.arange(pad, dtype=jnp.int32)
    src_p = jnp.concatenate([src, ar % _N])
    dst_p = jnp.concatenate([dst, _N + ar % _NDUMMY])
    se_p = (src_p | (dst_p << 16)).reshape(_NW, _NB, _B)
    dst_deg = dst_p.reshape(_NW, _NBD, _DB)

    deg_parts = _deg_kernel(dst_deg)
    hp, d = _tc_k1(x, W1, deg_parts.T[:_N])
    y1 = _scat_kernel(hp, se_p)
    hp2 = _tc_mid(y1, hp, d, b1.reshape(1, _D), W2)
    y2 = _scat_kernel(hp2, se_p)
    emb, logits = _tc_fin(y2, hp2, d, b2.reshape(1, _D), Wc, bc.reshape(1, _NCLS))
    return emb, logits
